# baseline (speedup 1.0000x reference)
import functools

import jax
import jax.numpy as jnp
from jax import lax
from jax.experimental import pallas as pl
from jax.experimental.pallas import tpu as pltpu
from jax.experimental.pallas import tpu_sc as plsc

N = 10000
NPAD = 10240
E = 320000
NC, NS = 2, 16
D_HID = 16
EPT = E // (NC * NS)
CH = 80
NCH = EPT // CH
NBUF = 5
RPT = NPAD // NS
NGRP = RPT // 16

_mesh = plsc.VectorSubcoreMesh(
    core_axis_name="c", subcore_axis_name="s", num_cores=NC, num_subcores=NS)


def _fill_1d(ref, n, value):
    v = jnp.full((16,), value, jnp.float32)

    def body(i, _):
        ref[pl.ds(i * 16, 16)] = v
        return 0

    lax.fori_loop(0, n // 16, body, 0)


def _fill_2d(ref, n, value):
    v = jnp.full((16,), value, jnp.float32)

    def body(i, _):
        ref[i, :] = v
        return 0

    lax.fori_loop(0, n, body, 0)


def _fast_rsqrt(v):
    i = lax.bitcast_convert_type(v, jnp.int32)
    i = jnp.int32(0x5F3759DF) - lax.shift_right_arithmetic(i, 1)
    y = lax.bitcast_convert_type(i, jnp.float32)
    for _ in range(3):
        y = y * (1.5 - 0.5 * v * y * y)
    return y


def _edge_loop(q_sp, acc_sp, src_v, dst_v, rows, gsems, ssems):
    for b in range(NBUF):
        pltpu.async_copy(q_sp.at[src_v.at[pl.ds(b * CH, CH)]], rows[b], gsems[b])

    def eloop(g, _):
        for b in range(NBUF):
            j = g * NBUF + b
            pltpu.make_async_copy(q_sp.at[src_v.at[pl.ds(pl.multiple_of(j * CH, 8), CH)]], rows[b], gsems[b]).wait()
            pltpu.async_copy(rows[b], acc_sp.at[dst_v.at[pl.ds(pl.multiple_of(j * CH, 8), CH)]], ssems[b], add=True)
        for b in range(NBUF):
            j = g * NBUF + b
            pltpu.make_async_copy(rows[b], acc_sp.at[dst_v.at[pl.ds(pl.multiple_of(j * CH, 8), CH)]], ssems[b]).wait()
            nj = j + NBUF

            @pl.when(nj < NCH)
            def _():
                pltpu.async_copy(q_sp.at[src_v.at[pl.ds(pl.multiple_of(nj * CH, 8), CH)]], rows[b], gsems[b])
        return 0

    lax.fori_loop(0, NCH // NBUF, eloop, 0)


@functools.partial(
    pl.kernel,
    out_type=(
        jax.ShapeDtypeStruct((NC, NPAD), jnp.float32),
        jax.ShapeDtypeStruct((NC, NPAD), jnp.float32),
    ),
    mesh=_mesh,
    scratch_types=[
        pltpu.VMEM((EPT,), jnp.int32),
        pltpu.VMEM((EPT,), jnp.int32),
        pltpu.VMEM((CH,), jnp.float32),
        pltpu.VMEM((CH,), jnp.float32),
        pltpu.SemaphoreType.DMA,
        pltpu.SemaphoreType.DMA,
        pltpu.VMEM_SHARED((NPAD,), jnp.float32),
        pltpu.VMEM_SHARED((NPAD,), jnp.float32),
    ],
    compiler_params=pltpu.CompilerParams(use_tc_tiling_on_sc=False, needs_layout_passes=False),
)
def _deg_kernel(srcr, dstr, dego_out, degi_out,
                src_v, dst_v, ones_v, zb_v, sem0, sem1, dego_sp, degi_sp):
    c = lax.axis_index("c")
    s = lax.axis_index("s")
    _fill_1d(ones_v, CH, 1.0)
    _fill_1d(zb_v, CH, 0.0)

    base = pl.multiple_of(s * RPT, 8)

    def zloop(k, _):
        off = pl.multiple_of(base + k * CH, 8)
        pltpu.sync_copy(zb_v, dego_sp.at[pl.ds(off, CH)])
        pltpu.sync_copy(zb_v, degi_sp.at[pl.ds(off, CH)])
        return 0

    lax.fori_loop(0, RPT // CH, zloop, 0)
    plsc.subcore_barrier()

    ebase = pl.multiple_of((c * NS + s) * EPT, 8)
    pltpu.sync_copy(srcr.at[pl.ds(ebase, EPT)], src_v)
    pltpu.sync_copy(dstr.at[pl.ds(ebase, EPT)], dst_v)

    def sloop(j, _):
        pltpu.async_copy(ones_v, dego_sp.at[src_v.at[pl.ds(pl.multiple_of(j * CH, 8), CH)]], sem0, add=True)
        pltpu.async_copy(ones_v, degi_sp.at[dst_v.at[pl.ds(pl.multiple_of(j * CH, 8), CH)]], sem1, add=True)
        return 0

    lax.fori_loop(0, NCH, sloop, 0)

    def dloop(j, _):
        pltpu.make_async_copy(ones_v, dego_sp.at[src_v.at[pl.ds(pl.multiple_of(j * CH, 8), CH)]], sem0).wait()
        pltpu.make_async_copy(ones_v, degi_sp.at[dst_v.at[pl.ds(pl.multiple_of(j * CH, 8), CH)]], sem1).wait()
        return 0

    lax.fori_loop(0, NCH, dloop, 0)
    plsc.subcore_barrier()

    pltpu.sync_copy(dego_sp.at[pl.ds(base, RPT)], dego_out.at[c, pl.ds(base, RPT)])
    pltpu.sync_copy(degi_sp.at[pl.ds(base, RPT)], degi_out.at[c, pl.ds(base, RPT)])


@functools.partial(
    pl.kernel,
    out_type=(
        jax.ShapeDtypeStruct((NC, NPAD, D_HID), jnp.float32),
        jax.ShapeDtypeStruct((NC, NPAD, D_HID), jnp.float32),
    ),
    mesh=_mesh,
    scratch_types=[
        pltpu.VMEM((EPT,), jnp.int32),
        pltpu.VMEM((EPT,), jnp.int32),
        [pltpu.VMEM((CH, D_HID), jnp.float32)] * NBUF,
        pltpu.VMEM((CH, D_HID), jnp.float32),
        pltpu.VMEM((RPT, D_HID), jnp.float32),
        pltpu.VMEM((RPT,), jnp.float32),
        pltpu.VMEM((RPT,), jnp.float32),
        [pltpu.SemaphoreType.DMA] * NBUF,
        [pltpu.SemaphoreType.DMA] * NBUF,
        pltpu.VMEM_SHARED((NPAD, D_HID), jnp.float32),
    ],
    compiler_params=pltpu.CompilerParams(use_tc_tiling_on_sc=False, needs_layout_passes=False),
)
def _agg1_kernel(p_hbm, dego, srcr, dstr, out_hbm, q_hbm,
                 src_v, dst_v, rows, zb_v, pv, da, db, gsems, ssems,
                 acc_sp):
    c = lax.axis_index("c")
    s = lax.axis_index("s")
    _fill_2d(zb_v, CH, 0.0)

    base = pl.multiple_of(s * RPT, 8)

    def zloop(k, _):
        off = pl.multiple_of(base + k * CH, 8)
        pltpu.sync_copy(zb_v, acc_sp.at[pl.ds(off, CH)])
        return 0

    lax.fori_loop(0, RPT // CH, zloop, 0)

    pltpu.sync_copy(p_hbm.at[pl.ds(base, RPT)], pv)
    pltpu.sync_copy(dego.at[0, pl.ds(base, RPT)], da)
    pltpu.sync_copy(dego.at[1, pl.ds(base, RPT)], db)

    iota16 = lax.iota(jnp.int32, 16)

    def scale(g, _):
        o = g * 16
        ns16 = _fast_rsqrt(jnp.maximum(da[pl.ds(o, 16)] + db[pl.ds(o, 16)], 1.0))
        rows16 = o + iota16
        for f in range(D_HID):
            fidx = jnp.full((16,), f, jnp.int32)
            col = plsc.load_gather(pv, [rows16, fidx])
            plsc.store_scatter(pv, [rows16, fidx], col * ns16)
        return 0

    lax.fori_loop(0, NGRP, scale, 0)
    pltpu.sync_copy(pv, q_hbm.at[c, pl.ds(base, RPT)])
    plsc.subcore_barrier()

    ebase = pl.multiple_of((c * NS + s) * EPT, 8)
    pltpu.sync_copy(srcr.at[pl.ds(ebase, EPT)], src_v)
    pltpu.sync_copy(dstr.at[pl.ds(ebase, EPT)], dst_v)
    _edge_loop(q_hbm.at[c], acc_sp, src_v, dst_v, rows, gsems, ssems)
    plsc.subcore_barrier()

    pltpu.sync_copy(acc_sp.at[pl.ds(base, RPT)], out_hbm.at[c, pl.ds(base, RPT)])


@functools.partial(
    pl.kernel,
    out_type=(
        jax.ShapeDtypeStruct((NC, NPAD, D_HID), jnp.float32),
        jax.ShapeDtypeStruct((NC, NPAD, D_HID), jnp.float32),
    ),
    mesh=_mesh,
    scratch_types=[
        pltpu.VMEM((EPT,), jnp.int32),
        pltpu.VMEM((EPT,), jnp.int32),
        [pltpu.VMEM((CH, D_HID), jnp.float32)] * NBUF,
        pltpu.VMEM((CH, D_HID), jnp.float32),
        pltpu.VMEM((RPT, D_HID), jnp.float32),
        pltpu.VMEM((RPT, D_HID), jnp.float32),
        pltpu.VMEM((RPT,), jnp.float32),
        pltpu.VMEM((RPT,), jnp.float32),
        pltpu.VMEM((RPT,), jnp.float32),
        pltpu.VMEM((16,), jnp.float32),
        [pltpu.SemaphoreType.DMA] * NBUF,
        [pltpu.SemaphoreType.DMA] * NBUF,
        pltpu.VMEM_SHARED((NPAD, D_HID), jnp.float32),
    ],
    compiler_params=pltpu.CompilerParams(use_tc_tiling_on_sc=False, needs_layout_passes=False),
)
def _agg2_kernel(p1, dego, degi, b1, srcr, dstr, out_hbm, q_hbm,
                 src_v, dst_v, rows, zb_v, p0v, p1v, da, db, dc, b1v,
                 gsems, ssems, acc_sp):
    c = lax.axis_index("c")
    s = lax.axis_index("s")
    _fill_2d(zb_v, CH, 0.0)

    base = pl.multiple_of(s * RPT, 8)

    def zloop(k, _):
        off = pl.multiple_of(base + k * CH, 8)
        pltpu.sync_copy(zb_v, acc_sp.at[pl.ds(off, CH)])
        return 0

    lax.fori_loop(0, RPT // CH, zloop, 0)

    pltpu.sync_copy(p1.at[0, pl.ds(base, RPT)], p0v)
    pltpu.sync_copy(p1.at[1, pl.ds(base, RPT)], p1v)
    pltpu.sync_copy(b1, b1v)

    pltpu.sync_copy(dego.at[0, pl.ds(base, RPT)], da)
    pltpu.sync_copy(dego.at[1, pl.ds(base, RPT)], dc)

    def nsl(i, _):
        o = i * 16
        da[pl.ds(o, 16)] = _fast_rsqrt(
            jnp.maximum(da[pl.ds(o, 16)] + dc[pl.ds(o, 16)], 1.0))
        return 0

    lax.fori_loop(0, RPT // 16, nsl, 0)
    pltpu.sync_copy(degi.at[0, pl.ds(base, RPT)], db)
    pltpu.sync_copy(degi.at[1, pl.ds(base, RPT)], dc)

    def ndl(i, _):
        o = i * 16
        db[pl.ds(o, 16)] = _fast_rsqrt(
            jnp.maximum(db[pl.ds(o, 16)] + dc[pl.ds(o, 16)], 1.0))
        return 0

    lax.fori_loop(0, RPT // 16, ndl, 0)

    iota16 = lax.iota(jnp.int32, 16)

    def mid(g, _):
        o = g * 16
        ns16 = da[pl.ds(o, 16)]
        nd16 = db[pl.ds(o, 16)]
        rows16 = o + iota16
        for f in range(D_HID):
            fidx = jnp.full((16,), f, jnp.int32)
            g0 = plsc.load_gather(p0v, [rows16, fidx])
            g1 = plsc.load_gather(p1v, [rows16, fidx])
            bf = plsc.load_gather(b1v, [fidx])
            v = jnp.maximum((g0 + g1) * nd16 + bf, 0.0) * ns16
            plsc.store_scatter(p0v, [rows16, fidx], v)
        return 0

    lax.fori_loop(0, NGRP, mid, 0)
    pltpu.sync_copy(p0v, q_hbm.at[c, pl.ds(base, RPT)])
    plsc.subcore_barrier()

    ebase = pl.multiple_of((c * NS + s) * EPT, 8)
    pltpu.sync_copy(srcr.at[pl.ds(ebase, EPT)], src_v)
    pltpu.sync_copy(dstr.at[pl.ds(ebase, EPT)], dst_v)
    _edge_loop(q_hbm.at[c], acc_sp, src_v, dst_v, rows, gsems, ssems)
    plsc.subcore_barrier()

    pltpu.sync_copy(acc_sp.at[pl.ds(base, RPT)], out_hbm.at[c, pl.ds(base, RPT)])


def _p_body(x_ref, w_ref, ei_ref, o_ref, os_ref, od_ref):
    p = jnp.dot(x_ref[...], w_ref[...], preferred_element_type=jnp.float32)
    o_ref[...] = jnp.concatenate(
        [p, jnp.zeros((NPAD - N, D_HID), jnp.float32)], axis=0)
    os_ref[...] = ei_ref[0]
    od_ref[...] = ei_ref[1]


_p_call = pl.pallas_call(
    _p_body,
    out_shape=(
        jax.ShapeDtypeStruct((NPAD, D_HID), jnp.float32),
        jax.ShapeDtypeStruct((E,), jnp.int32),
        jax.ShapeDtypeStruct((E,), jnp.int32),
    ))


def _out_body(p_ref, degi_ref, w_ref, b_ref, o_ref):
    agg = p_ref[0] + p_ref[1]
    nd = lax.rsqrt(jnp.maximum(degi_ref[0] + degi_ref[1], 1.0))
    o_ref[...] = jnp.dot(agg * nd[:, None], w_ref[...],
                         preferred_element_type=jnp.float32) + b_ref[...][None, :]


_out_call = pl.pallas_call(
    _out_body, out_shape=jax.ShapeDtypeStruct((NPAD, 40), jnp.float32))


def kernel(x, edge_index, W1, b1, W2, b2):
    ei = edge_index.astype(jnp.int32)

    P, srcp, dstp = _p_call(x, W1, ei)
    dego, degi = _deg_kernel(srcp, dstp)
    p1, _ = _agg1_kernel(P, dego, srcp, dstp)
    p2, _ = _agg2_kernel(p1, dego, degi, b1, srcp, dstp)
    out = _out_call(p2, degi, W2, b2)
    return out[:N]


# speedup vs baseline: 1.0861x; 1.0861x over previous
import functools

import jax
import jax.numpy as jnp
from jax import lax
from jax.experimental import pallas as pl
from jax.experimental.pallas import tpu as pltpu
from jax.experimental.pallas import tpu_sc as plsc

N = 10000
NPAD = 10240
E = 320000
NC, NS = 2, 16
D_HID = 16
EPT = E // (NC * NS)
CH = 80
NCH = EPT // CH
NBUF = 5
RPT = NPAD // NS
NGRP = RPT // 16

_mesh = plsc.VectorSubcoreMesh(
    core_axis_name="c", subcore_axis_name="s", num_cores=NC, num_subcores=NS)


def _fill_1d(ref, n, value):
    v = jnp.full((16,), value, jnp.float32)

    def body(i, _):
        ref[pl.ds(i * 16, 16)] = v
        return 0

    lax.fori_loop(0, n // 16, body, 0)


def _fill_2d(ref, n, value):
    v = jnp.full((16,), value, jnp.float32)

    def body(i, _):
        ref[i, :] = v
        return 0

    lax.fori_loop(0, n, body, 0)


def _fast_rsqrt(v):
    i = lax.bitcast_convert_type(v, jnp.int32)
    i = jnp.int32(0x5F3759DF) - lax.shift_right_arithmetic(i, 1)
    y = lax.bitcast_convert_type(i, jnp.float32)
    for _ in range(3):
        y = y * (1.5 - 0.5 * v * y * y)
    return y


def _edge_loop(q_sp, acc_sp, src_v, dst_v, rows, gsems, ssems):
    for b in range(NBUF):
        pltpu.async_copy(q_sp.at[src_v.at[pl.ds(b * CH, CH)]], rows[b], gsems[b])

    def eloop(g, _):
        for b in range(NBUF):
            j = g * NBUF + b
            pltpu.make_async_copy(q_sp.at[src_v.at[pl.ds(pl.multiple_of(j * CH, 8), CH)]], rows[b], gsems[b]).wait()
            pltpu.async_copy(rows[b], acc_sp.at[dst_v.at[pl.ds(pl.multiple_of(j * CH, 8), CH)]], ssems[b], add=True)
        for b in range(NBUF):
            j = g * NBUF + b
            pltpu.make_async_copy(rows[b], acc_sp.at[dst_v.at[pl.ds(pl.multiple_of(j * CH, 8), CH)]], ssems[b]).wait()
            nj = j + NBUF

            @pl.when(nj < NCH)
            def _():
                pltpu.async_copy(q_sp.at[src_v.at[pl.ds(pl.multiple_of(nj * CH, 8), CH)]], rows[b], gsems[b])
        return 0

    lax.fori_loop(0, NCH // NBUF, eloop, 0)


@functools.partial(
    pl.kernel,
    out_type=(
        jax.ShapeDtypeStruct((NC, NPAD), jnp.float32),
        jax.ShapeDtypeStruct((NC, NPAD), jnp.float32),
    ),
    mesh=_mesh,
    scratch_types=[
        pltpu.VMEM((EPT,), jnp.int32),
        pltpu.VMEM((EPT,), jnp.int32),
        pltpu.VMEM((CH,), jnp.float32),
        pltpu.VMEM((CH,), jnp.float32),
        pltpu.SemaphoreType.DMA,
        pltpu.SemaphoreType.DMA,
        pltpu.VMEM_SHARED((NPAD,), jnp.float32),
        pltpu.VMEM_SHARED((NPAD,), jnp.float32),
    ],
    compiler_params=pltpu.CompilerParams(use_tc_tiling_on_sc=False, needs_layout_passes=False),
)
def _deg_kernel(srcr, dstr, dego_out, degi_out,
                src_v, dst_v, ones_v, zb_v, sem0, sem1, dego_sp, degi_sp):
    c = lax.axis_index("c")
    s = lax.axis_index("s")
    _fill_1d(ones_v, CH, 1.0)
    _fill_1d(zb_v, CH, 0.0)

    base = pl.multiple_of(s * RPT, 8)

    def zloop(k, _):
        off = pl.multiple_of(base + k * CH, 8)
        pltpu.sync_copy(zb_v, dego_sp.at[pl.ds(off, CH)])
        pltpu.sync_copy(zb_v, degi_sp.at[pl.ds(off, CH)])
        return 0

    lax.fori_loop(0, RPT // CH, zloop, 0)
    plsc.subcore_barrier()

    ebase = pl.multiple_of((c * NS + s) * EPT, 8)
    pltpu.sync_copy(srcr.at[pl.ds(ebase, EPT)], src_v)
    pltpu.sync_copy(dstr.at[pl.ds(ebase, EPT)], dst_v)

    def sloop(j, _):
        pltpu.async_copy(ones_v, dego_sp.at[src_v.at[pl.ds(pl.multiple_of(j * CH, 8), CH)]], sem0, add=True)
        pltpu.async_copy(ones_v, degi_sp.at[dst_v.at[pl.ds(pl.multiple_of(j * CH, 8), CH)]], sem1, add=True)
        return 0

    lax.fori_loop(0, NCH, sloop, 0)

    def dloop(j, _):
        pltpu.make_async_copy(ones_v, dego_sp.at[src_v.at[pl.ds(pl.multiple_of(j * CH, 8), CH)]], sem0).wait()
        pltpu.make_async_copy(ones_v, degi_sp.at[dst_v.at[pl.ds(pl.multiple_of(j * CH, 8), CH)]], sem1).wait()
        return 0

    lax.fori_loop(0, NCH, dloop, 0)
    plsc.subcore_barrier()

    pltpu.sync_copy(dego_sp.at[pl.ds(base, RPT)], dego_out.at[c, pl.ds(base, RPT)])
    pltpu.sync_copy(degi_sp.at[pl.ds(base, RPT)], degi_out.at[c, pl.ds(base, RPT)])


@functools.partial(
    pl.kernel,
    out_type=jax.ShapeDtypeStruct((NC, NPAD, D_HID), jnp.float32),
    mesh=_mesh,
    scratch_types=[
        pltpu.VMEM((EPT,), jnp.int32),
        pltpu.VMEM((EPT,), jnp.int32),
        [pltpu.VMEM((CH, D_HID), jnp.float32)] * NBUF,
        pltpu.VMEM((CH, D_HID), jnp.float32),
        pltpu.VMEM((RPT, D_HID), jnp.float32),
        pltpu.VMEM((RPT,), jnp.float32),
        pltpu.VMEM((RPT,), jnp.float32),
        [pltpu.SemaphoreType.DMA] * NBUF,
        [pltpu.SemaphoreType.DMA] * NBUF,
        pltpu.VMEM_SHARED((NPAD, D_HID), jnp.float32),
        pltpu.VMEM_SHARED((NPAD, D_HID), jnp.float32),
    ],
    compiler_params=pltpu.CompilerParams(use_tc_tiling_on_sc=False, needs_layout_passes=False),
)
def _agg1_kernel(p_hbm, dego, srcr, dstr, out_hbm,
                 src_v, dst_v, rows, zb_v, pv, da, db, gsems, ssems,
                 q_sp, acc_sp):
    c = lax.axis_index("c")
    s = lax.axis_index("s")
    _fill_2d(zb_v, CH, 0.0)

    base = pl.multiple_of(s * RPT, 8)

    def zloop(k, _):
        off = pl.multiple_of(base + k * CH, 8)
        pltpu.sync_copy(zb_v, acc_sp.at[pl.ds(off, CH)])
        return 0

    lax.fori_loop(0, RPT // CH, zloop, 0)

    pltpu.sync_copy(p_hbm.at[pl.ds(base, RPT)], pv)
    pltpu.sync_copy(dego.at[0, pl.ds(base, RPT)], da)
    pltpu.sync_copy(dego.at[1, pl.ds(base, RPT)], db)

    iota16 = lax.iota(jnp.int32, 16)

    def scale(g, _):
        o = g * 16
        ns16 = _fast_rsqrt(jnp.maximum(da[pl.ds(o, 16)] + db[pl.ds(o, 16)], 1.0))
        rows16 = o + iota16
        for f in range(D_HID):
            fidx = jnp.full((16,), f, jnp.int32)
            col = plsc.load_gather(pv, [rows16, fidx])
            plsc.store_scatter(pv, [rows16, fidx], col * ns16)
        return 0

    lax.fori_loop(0, NGRP, scale, 0)
    pltpu.sync_copy(pv, q_sp.at[pl.ds(base, RPT)])
    plsc.subcore_barrier()

    ebase = pl.multiple_of((c * NS + s) * EPT, 8)
    pltpu.sync_copy(srcr.at[pl.ds(ebase, EPT)], src_v)
    pltpu.sync_copy(dstr.at[pl.ds(ebase, EPT)], dst_v)
    _edge_loop(q_sp, acc_sp, src_v, dst_v, rows, gsems, ssems)
    plsc.subcore_barrier()

    pltpu.sync_copy(acc_sp.at[pl.ds(base, RPT)], out_hbm.at[c, pl.ds(base, RPT)])


@functools.partial(
    pl.kernel,
    out_type=jax.ShapeDtypeStruct((NC, NPAD, D_HID), jnp.float32),
    mesh=_mesh,
    scratch_types=[
        pltpu.VMEM((EPT,), jnp.int32),
        pltpu.VMEM((EPT,), jnp.int32),
        [pltpu.VMEM((CH, D_HID), jnp.float32)] * NBUF,
        pltpu.VMEM((CH, D_HID), jnp.float32),
        pltpu.VMEM((RPT, D_HID), jnp.float32),
        pltpu.VMEM((RPT, D_HID), jnp.float32),
        pltpu.VMEM((RPT,), jnp.float32),
        pltpu.VMEM((RPT,), jnp.float32),
        pltpu.VMEM((RPT,), jnp.float32),
        pltpu.VMEM((16,), jnp.float32),
        [pltpu.SemaphoreType.DMA] * NBUF,
        [pltpu.SemaphoreType.DMA] * NBUF,
        pltpu.VMEM_SHARED((NPAD, D_HID), jnp.float32),
        pltpu.VMEM_SHARED((NPAD, D_HID), jnp.float32),
    ],
    compiler_params=pltpu.CompilerParams(use_tc_tiling_on_sc=False, needs_layout_passes=False),
)
def _agg2_kernel(p1, dego, degi, b1, srcr, dstr, out_hbm,
                 src_v, dst_v, rows, zb_v, p0v, p1v, da, db, dc, b1v,
                 gsems, ssems, q_sp, acc_sp):
    c = lax.axis_index("c")
    s = lax.axis_index("s")
    _fill_2d(zb_v, CH, 0.0)

    base = pl.multiple_of(s * RPT, 8)

    def zloop(k, _):
        off = pl.multiple_of(base + k * CH, 8)
        pltpu.sync_copy(zb_v, acc_sp.at[pl.ds(off, CH)])
        return 0

    lax.fori_loop(0, RPT // CH, zloop, 0)

    pltpu.sync_copy(p1.at[0, pl.ds(base, RPT)], p0v)
    pltpu.sync_copy(p1.at[1, pl.ds(base, RPT)], p1v)
    pltpu.sync_copy(b1, b1v)

    pltpu.sync_copy(dego.at[0, pl.ds(base, RPT)], da)
    pltpu.sync_copy(dego.at[1, pl.ds(base, RPT)], dc)

    def nsl(i, _):
        o = i * 16
        da[pl.ds(o, 16)] = _fast_rsqrt(
            jnp.maximum(da[pl.ds(o, 16)] + dc[pl.ds(o, 16)], 1.0))
        return 0

    lax.fori_loop(0, RPT // 16, nsl, 0)
    pltpu.sync_copy(degi.at[0, pl.ds(base, RPT)], db)
    pltpu.sync_copy(degi.at[1, pl.ds(base, RPT)], dc)

    def ndl(i, _):
        o = i * 16
        db[pl.ds(o, 16)] = _fast_rsqrt(
            jnp.maximum(db[pl.ds(o, 16)] + dc[pl.ds(o, 16)], 1.0))
        return 0

    lax.fori_loop(0, RPT // 16, ndl, 0)

    iota16 = lax.iota(jnp.int32, 16)

    def mid(g, _):
        o = g * 16
        ns16 = da[pl.ds(o, 16)]
        nd16 = db[pl.ds(o, 16)]
        rows16 = o + iota16
        for f in range(D_HID):
            fidx = jnp.full((16,), f, jnp.int32)
            g0 = plsc.load_gather(p0v, [rows16, fidx])
            g1 = plsc.load_gather(p1v, [rows16, fidx])
            bf = plsc.load_gather(b1v, [fidx])
            v = jnp.maximum((g0 + g1) * nd16 + bf, 0.0) * ns16
            plsc.store_scatter(p0v, [rows16, fidx], v)
        return 0

    lax.fori_loop(0, NGRP, mid, 0)
    pltpu.sync_copy(p0v, q_sp.at[pl.ds(base, RPT)])
    plsc.subcore_barrier()

    ebase = pl.multiple_of((c * NS + s) * EPT, 8)
    pltpu.sync_copy(srcr.at[pl.ds(ebase, EPT)], src_v)
    pltpu.sync_copy(dstr.at[pl.ds(ebase, EPT)], dst_v)
    _edge_loop(q_sp, acc_sp, src_v, dst_v, rows, gsems, ssems)
    plsc.subcore_barrier()

    pltpu.sync_copy(acc_sp.at[pl.ds(base, RPT)], out_hbm.at[c, pl.ds(base, RPT)])


def _p_body(x_ref, w_ref, ei_ref, o_ref, os_ref, od_ref):
    p = jnp.dot(x_ref[...], w_ref[...], preferred_element_type=jnp.float32)
    o_ref[...] = jnp.concatenate(
        [p, jnp.zeros((NPAD - N, D_HID), jnp.float32)], axis=0)
    os_ref[...] = ei_ref[0]
    od_ref[...] = ei_ref[1]


_p_call = pl.pallas_call(
    _p_body,
    out_shape=(
        jax.ShapeDtypeStruct((NPAD, D_HID), jnp.float32),
        jax.ShapeDtypeStruct((E,), jnp.int32),
        jax.ShapeDtypeStruct((E,), jnp.int32),
    ))


def _out_body(p_ref, degi_ref, w_ref, b_ref, o_ref):
    agg = p_ref[0] + p_ref[1]
    nd = lax.rsqrt(jnp.maximum(degi_ref[0] + degi_ref[1], 1.0))
    o_ref[...] = jnp.dot(agg * nd[:, None], w_ref[...],
                         preferred_element_type=jnp.float32) + b_ref[...][None, :]


_out_call = pl.pallas_call(
    _out_body, out_shape=jax.ShapeDtypeStruct((NPAD, 40), jnp.float32))


def kernel(x, edge_index, W1, b1, W2, b2):
    ei = edge_index.astype(jnp.int32)

    P, srcp, dstp = _p_call(x, W1, ei)
    dego, degi = _deg_kernel(srcp, dstp)
    p1 = _agg1_kernel(P, dego, srcp, dstp)
    p2 = _agg2_kernel(p1, dego, degi, b1, srcp, dstp)
    out = _out_call(p2, degi, W2, b2)
    return out[:N]


# speedup vs baseline: 1.0889x; 1.0026x over previous
import functools

import jax
import jax.numpy as jnp
from jax import lax
from jax.experimental import pallas as pl
from jax.experimental.pallas import tpu as pltpu
from jax.experimental.pallas import tpu_sc as plsc

N = 10000
NPAD = 10240
E = 320000
NC, NS = 2, 16
D_HID = 16
EPT = E // (NC * NS)
CH = 80
NCH = EPT // CH
NBUF = 5
RPT = NPAD // NS
DCH = 128
DNF = EPT // DCH
DTL = EPT - DNF * DCH
NGRP = RPT // 16

_mesh = plsc.VectorSubcoreMesh(
    core_axis_name="c", subcore_axis_name="s", num_cores=NC, num_subcores=NS)


def _fill_1d(ref, n, value):
    v = jnp.full((16,), value, jnp.float32)

    def body(i, _):
        ref[pl.ds(i * 16, 16)] = v
        return 0

    lax.fori_loop(0, n // 16, body, 0)


def _fill_2d(ref, n, value):
    v = jnp.full((16,), value, jnp.float32)

    def body(i, _):
        ref[i, :] = v
        return 0

    lax.fori_loop(0, n, body, 0)


def _fast_rsqrt(v):
    i = lax.bitcast_convert_type(v, jnp.int32)
    i = jnp.int32(0x5F3759DF) - lax.shift_right_arithmetic(i, 1)
    y = lax.bitcast_convert_type(i, jnp.float32)
    for _ in range(3):
        y = y * (1.5 - 0.5 * v * y * y)
    return y


def _edge_loop(q_sp, acc_sp, src_v, dst_v, rows, gsems, ssems):
    for b in range(NBUF):
        pltpu.async_copy(q_sp.at[src_v.at[pl.ds(b * CH, CH)]], rows[b], gsems[b])

    def eloop(g, _):
        for b in range(NBUF):
            j = g * NBUF + b
            pltpu.make_async_copy(q_sp.at[src_v.at[pl.ds(pl.multiple_of(j * CH, 8), CH)]], rows[b], gsems[b]).wait()
            pltpu.async_copy(rows[b], acc_sp.at[dst_v.at[pl.ds(pl.multiple_of(j * CH, 8), CH)]], ssems[b], add=True)
        for b in range(NBUF):
            j = g * NBUF + b
            pltpu.make_async_copy(rows[b], acc_sp.at[dst_v.at[pl.ds(pl.multiple_of(j * CH, 8), CH)]], ssems[b]).wait()
            nj = j + NBUF

            @pl.when(nj < NCH)
            def _():
                pltpu.async_copy(q_sp.at[src_v.at[pl.ds(pl.multiple_of(nj * CH, 8), CH)]], rows[b], gsems[b])
        return 0

    lax.fori_loop(0, NCH // NBUF, eloop, 0)


@functools.partial(
    pl.kernel,
    out_type=(
        jax.ShapeDtypeStruct((NC, NPAD), jnp.float32),
        jax.ShapeDtypeStruct((NC, NPAD), jnp.float32),
    ),
    mesh=_mesh,
    scratch_types=[
        pltpu.VMEM((EPT,), jnp.int32),
        pltpu.VMEM((EPT,), jnp.int32),
        pltpu.VMEM((DCH,), jnp.float32),
        pltpu.VMEM((DCH,), jnp.float32),
        pltpu.SemaphoreType.DMA,
        pltpu.SemaphoreType.DMA,
        pltpu.VMEM_SHARED((NPAD,), jnp.float32),
        pltpu.VMEM_SHARED((NPAD,), jnp.float32),
    ],
    compiler_params=pltpu.CompilerParams(use_tc_tiling_on_sc=False, needs_layout_passes=False),
)
def _deg_kernel(srcr, dstr, dego_out, degi_out,
                src_v, dst_v, ones_v, zb_v, sem0, sem1, dego_sp, degi_sp):
    c = lax.axis_index("c")
    s = lax.axis_index("s")
    _fill_1d(ones_v, DCH, 1.0)
    _fill_1d(zb_v, DCH, 0.0)

    base = pl.multiple_of(s * RPT, 8)

    def zloop(k, _):
        off = pl.multiple_of(base + k * DCH, 8)
        pltpu.sync_copy(zb_v, dego_sp.at[pl.ds(off, DCH)])
        pltpu.sync_copy(zb_v, degi_sp.at[pl.ds(off, DCH)])
        return 0

    lax.fori_loop(0, RPT // DCH, zloop, 0)
    plsc.subcore_barrier()

    ebase = pl.multiple_of((c * NS + s) * EPT, 8)
    pltpu.sync_copy(srcr.at[pl.ds(ebase, EPT)], src_v)
    pltpu.sync_copy(dstr.at[pl.ds(ebase, EPT)], dst_v)

    def sloop(j, _):
        pltpu.async_copy(ones_v, dego_sp.at[src_v.at[pl.ds(pl.multiple_of(j * DCH, 8), DCH)]], sem0, add=True)
        pltpu.async_copy(ones_v, degi_sp.at[dst_v.at[pl.ds(pl.multiple_of(j * DCH, 8), DCH)]], sem1, add=True)
        return 0

    lax.fori_loop(0, DNF, sloop, 0)
    ot = ones_v.at[pl.ds(0, DTL)]
    pltpu.async_copy(ot, dego_sp.at[src_v.at[pl.ds(DNF * DCH, DTL)]], sem0, add=True)
    pltpu.async_copy(ot, degi_sp.at[dst_v.at[pl.ds(DNF * DCH, DTL)]], sem1, add=True)

    def dloop(j, _):
        pltpu.make_async_copy(ones_v, dego_sp.at[src_v.at[pl.ds(pl.multiple_of(j * DCH, 8), DCH)]], sem0).wait()
        pltpu.make_async_copy(ones_v, degi_sp.at[dst_v.at[pl.ds(pl.multiple_of(j * DCH, 8), DCH)]], sem1).wait()
        return 0

    lax.fori_loop(0, DNF, dloop, 0)
    pltpu.make_async_copy(ot, dego_sp.at[src_v.at[pl.ds(DNF * DCH, DTL)]], sem0).wait()
    pltpu.make_async_copy(ot, degi_sp.at[dst_v.at[pl.ds(DNF * DCH, DTL)]], sem1).wait()
    plsc.subcore_barrier()

    pltpu.sync_copy(dego_sp.at[pl.ds(base, RPT)], dego_out.at[c, pl.ds(base, RPT)])
    pltpu.sync_copy(degi_sp.at[pl.ds(base, RPT)], degi_out.at[c, pl.ds(base, RPT)])


@functools.partial(
    pl.kernel,
    out_type=jax.ShapeDtypeStruct((NC, NPAD, D_HID), jnp.float32),
    mesh=_mesh,
    scratch_types=[
        pltpu.VMEM((EPT,), jnp.int32),
        pltpu.VMEM((EPT,), jnp.int32),
        [pltpu.VMEM((CH, D_HID), jnp.float32)] * NBUF,
        pltpu.VMEM((CH, D_HID), jnp.float32),
        pltpu.VMEM((RPT, D_HID), jnp.float32),
        pltpu.VMEM((RPT,), jnp.float32),
        pltpu.VMEM((RPT,), jnp.float32),
        [pltpu.SemaphoreType.DMA] * NBUF,
        [pltpu.SemaphoreType.DMA] * NBUF,
        pltpu.VMEM_SHARED((NPAD, D_HID), jnp.float32),
        pltpu.VMEM_SHARED((NPAD, D_HID), jnp.float32),
    ],
    compiler_params=pltpu.CompilerParams(use_tc_tiling_on_sc=False, needs_layout_passes=False),
)
def _agg1_kernel(p_hbm, dego, srcr, dstr, out_hbm,
                 src_v, dst_v, rows, zb_v, pv, da, db, gsems, ssems,
                 q_sp, acc_sp):
    c = lax.axis_index("c")
    s = lax.axis_index("s")
    _fill_2d(zb_v, CH, 0.0)

    base = pl.multiple_of(s * RPT, 8)

    def zloop(k, _):
        off = pl.multiple_of(base + k * CH, 8)
        pltpu.sync_copy(zb_v, acc_sp.at[pl.ds(off, CH)])
        return 0

    lax.fori_loop(0, RPT // CH, zloop, 0)

    pltpu.sync_copy(p_hbm.at[pl.ds(base, RPT)], pv)
    pltpu.sync_copy(dego.at[0, pl.ds(base, RPT)], da)
    pltpu.sync_copy(dego.at[1, pl.ds(base, RPT)], db)

    iota16 = lax.iota(jnp.int32, 16)

    def scale(g, _):
        o = g * 16
        ns16 = _fast_rsqrt(jnp.maximum(da[pl.ds(o, 16)] + db[pl.ds(o, 16)], 1.0))
        rows16 = o + iota16
        for f in range(D_HID):
            fidx = jnp.full((16,), f, jnp.int32)
            col = plsc.load_gather(pv, [rows16, fidx])
            plsc.store_scatter(pv, [rows16, fidx], col * ns16)
        return 0

    lax.fori_loop(0, NGRP, scale, 0)
    pltpu.sync_copy(pv, q_sp.at[pl.ds(base, RPT)])
    plsc.subcore_barrier()

    ebase = pl.multiple_of((c * NS + s) * EPT, 8)
    pltpu.sync_copy(srcr.at[pl.ds(ebase, EPT)], src_v)
    pltpu.sync_copy(dstr.at[pl.ds(ebase, EPT)], dst_v)
    _edge_loop(q_sp, acc_sp, src_v, dst_v, rows, gsems, ssems)
    plsc.subcore_barrier()

    pltpu.sync_copy(acc_sp.at[pl.ds(base, RPT)], out_hbm.at[c, pl.ds(base, RPT)])


@functools.partial(
    pl.kernel,
    out_type=jax.ShapeDtypeStruct((NC, NPAD, D_HID), jnp.float32),
    mesh=_mesh,
    scratch_types=[
        pltpu.VMEM((EPT,), jnp.int32),
        pltpu.VMEM((EPT,), jnp.int32),
        [pltpu.VMEM((CH, D_HID), jnp.float32)] * NBUF,
        pltpu.VMEM((CH, D_HID), jnp.float32),
        pltpu.VMEM((RPT, D_HID), jnp.float32),
        pltpu.VMEM((RPT, D_HID), jnp.float32),
        pltpu.VMEM((RPT,), jnp.float32),
        pltpu.VMEM((RPT,), jnp.float32),
        pltpu.VMEM((RPT,), jnp.float32),
        pltpu.VMEM((16,), jnp.float32),
        [pltpu.SemaphoreType.DMA] * NBUF,
        [pltpu.SemaphoreType.DMA] * NBUF,
        pltpu.VMEM_SHARED((NPAD, D_HID), jnp.float32),
        pltpu.VMEM_SHARED((NPAD, D_HID), jnp.float32),
    ],
    compiler_params=pltpu.CompilerParams(use_tc_tiling_on_sc=False, needs_layout_passes=False),
)
def _agg2_kernel(p1, dego, degi, b1, srcr, dstr, out_hbm,
                 src_v, dst_v, rows, zb_v, p0v, p1v, da, db, dc, b1v,
                 gsems, ssems, q_sp, acc_sp):
    c = lax.axis_index("c")
    s = lax.axis_index("s")
    _fill_2d(zb_v, CH, 0.0)

    base = pl.multiple_of(s * RPT, 8)

    def zloop(k, _):
        off = pl.multiple_of(base + k * CH, 8)
        pltpu.sync_copy(zb_v, acc_sp.at[pl.ds(off, CH)])
        return 0

    lax.fori_loop(0, RPT // CH, zloop, 0)

    pltpu.sync_copy(p1.at[0, pl.ds(base, RPT)], p0v)
    pltpu.sync_copy(p1.at[1, pl.ds(base, RPT)], p1v)
    pltpu.sync_copy(b1, b1v)

    pltpu.sync_copy(dego.at[0, pl.ds(base, RPT)], da)
    pltpu.sync_copy(dego.at[1, pl.ds(base, RPT)], dc)

    def nsl(i, _):
        o = i * 16
        da[pl.ds(o, 16)] = _fast_rsqrt(
            jnp.maximum(da[pl.ds(o, 16)] + dc[pl.ds(o, 16)], 1.0))
        return 0

    lax.fori_loop(0, RPT // 16, nsl, 0)
    pltpu.sync_copy(degi.at[0, pl.ds(base, RPT)], db)
    pltpu.sync_copy(degi.at[1, pl.ds(base, RPT)], dc)

    def ndl(i, _):
        o = i * 16
        db[pl.ds(o, 16)] = _fast_rsqrt(
            jnp.maximum(db[pl.ds(o, 16)] + dc[pl.ds(o, 16)], 1.0))
        return 0

    lax.fori_loop(0, RPT // 16, ndl, 0)

    iota16 = lax.iota(jnp.int32, 16)
    bexp = [plsc.load_gather(b1v, [jnp.full((16,), f, jnp.int32)])
            for f in range(D_HID)]

    def mid(g, _):
        o = g * 16
        ns16 = da[pl.ds(o, 16)]
        nd16 = db[pl.ds(o, 16)]
        rows16 = o + iota16
        for f in range(D_HID):
            fidx = jnp.full((16,), f, jnp.int32)
            g0 = plsc.load_gather(p0v, [rows16, fidx])
            g1 = plsc.load_gather(p1v, [rows16, fidx])
            v = jnp.maximum((g0 + g1) * nd16 + bexp[f], 0.0) * ns16
            plsc.store_scatter(p0v, [rows16, fidx], v)
        return 0

    lax.fori_loop(0, NGRP, mid, 0)
    pltpu.sync_copy(p0v, q_sp.at[pl.ds(base, RPT)])
    plsc.subcore_barrier()

    ebase = pl.multiple_of((c * NS + s) * EPT, 8)
    pltpu.sync_copy(srcr.at[pl.ds(ebase, EPT)], src_v)
    pltpu.sync_copy(dstr.at[pl.ds(ebase, EPT)], dst_v)
    _edge_loop(q_sp, acc_sp, src_v, dst_v, rows, gsems, ssems)
    plsc.subcore_barrier()

    pltpu.sync_copy(acc_sp.at[pl.ds(base, RPT)], out_hbm.at[c, pl.ds(base, RPT)])


def _p_body(x_ref, w_ref, ei_ref, o_ref, os_ref, od_ref):
    p = jnp.dot(x_ref[...], w_ref[...], preferred_element_type=jnp.float32)
    o_ref[...] = jnp.concatenate(
        [p, jnp.zeros((NPAD - N, D_HID), jnp.float32)], axis=0)
    os_ref[...] = ei_ref[0]
    od_ref[...] = ei_ref[1]


_p_call = pl.pallas_call(
    _p_body,
    out_shape=(
        jax.ShapeDtypeStruct((NPAD, D_HID), jnp.float32),
        jax.ShapeDtypeStruct((E,), jnp.int32),
        jax.ShapeDtypeStruct((E,), jnp.int32),
    ))


def _out_body(p_ref, degi_ref, w_ref, b_ref, o_ref):
    agg = p_ref[0] + p_ref[1]
    nd = lax.rsqrt(jnp.maximum(degi_ref[0] + degi_ref[1], 1.0))
    o_ref[...] = jnp.dot(agg * nd[:, None], w_ref[...],
                         preferred_element_type=jnp.float32) + b_ref[...][None, :]


_out_call = pl.pallas_call(
    _out_body, out_shape=jax.ShapeDtypeStruct((NPAD, 40), jnp.float32))


def kernel(x, edge_index, W1, b1, W2, b2):
    ei = edge_index.astype(jnp.int32)

    P, srcp, dstp = _p_call(x, W1, ei)
    dego, degi = _deg_kernel(srcp, dstp)
    p1 = _agg1_kernel(P, dego, srcp, dstp)
    p2 = _agg2_kernel(p1, dego, degi, b1, srcp, dstp)
    out = _out_call(p2, degi, W2, b2)
    return out[:N]


# speedup vs baseline: 1.0973x; 1.0077x over previous
import functools

import jax
import jax.numpy as jnp
from jax import lax
from jax.experimental import pallas as pl
from jax.experimental.pallas import tpu as pltpu
from jax.experimental.pallas import tpu_sc as plsc

N = 10000
NPAD = 10240
E = 320000
NC, NS = 2, 16
D_HID = 16
EPT = E // (NC * NS)
CH = 80
NCH = EPT // CH
NBUF = 10
RPT = NPAD // NS
DCH = 128
DNF = EPT // DCH
DTL = EPT - DNF * DCH
NGRP = RPT // 16

_mesh = plsc.VectorSubcoreMesh(
    core_axis_name="c", subcore_axis_name="s", num_cores=NC, num_subcores=NS)


def _fill_1d(ref, n, value):
    v = jnp.full((16,), value, jnp.float32)

    def body(i, _):
        ref[pl.ds(i * 16, 16)] = v
        return 0

    lax.fori_loop(0, n // 16, body, 0)


def _fill_2d(ref, n, value):
    v = jnp.full((16,), value, jnp.float32)

    def body(i, _):
        ref[i, :] = v
        return 0

    lax.fori_loop(0, n, body, 0)


def _fast_rsqrt(v):
    i = lax.bitcast_convert_type(v, jnp.int32)
    i = jnp.int32(0x5F3759DF) - lax.shift_right_arithmetic(i, 1)
    y = lax.bitcast_convert_type(i, jnp.float32)
    for _ in range(3):
        y = y * (1.5 - 0.5 * v * y * y)
    return y


def _edge_loop(q_sp, acc_sp, src_v, dst_v, rows, gsems, ssems):
    ngrp = NCH // NBUF
    ntail = NCH - ngrp * NBUF
    for b in range(NBUF):
        pltpu.async_copy(q_sp.at[src_v.at[pl.ds(b * CH, CH)]], rows[b], gsems[b])

    def eloop(g, _):
        for b in range(NBUF):
            j = g * NBUF + b
            pltpu.make_async_copy(q_sp.at[src_v.at[pl.ds(pl.multiple_of(j * CH, 8), CH)]], rows[b], gsems[b]).wait()
            pltpu.async_copy(rows[b], acc_sp.at[dst_v.at[pl.ds(pl.multiple_of(j * CH, 8), CH)]], ssems[b], add=True)
        for b in range(NBUF):
            j = g * NBUF + b
            pltpu.make_async_copy(rows[b], acc_sp.at[dst_v.at[pl.ds(pl.multiple_of(j * CH, 8), CH)]], ssems[b]).wait()
            nj = j + NBUF

            @pl.when(nj < NCH)
            def _():
                pltpu.async_copy(q_sp.at[src_v.at[pl.ds(pl.multiple_of(nj * CH, 8), CH)]], rows[b], gsems[b])
        return 0

    lax.fori_loop(0, ngrp, eloop, 0)
    for b in range(ntail):
        j = ngrp * NBUF + b
        pltpu.make_async_copy(q_sp.at[src_v.at[pl.ds(j * CH, CH)]], rows[b], gsems[b]).wait()
        pltpu.async_copy(rows[b], acc_sp.at[dst_v.at[pl.ds(j * CH, CH)]], ssems[b], add=True)
    for b in range(ntail):
        j = ngrp * NBUF + b
        pltpu.make_async_copy(rows[b], acc_sp.at[dst_v.at[pl.ds(j * CH, CH)]], ssems[b]).wait()


@functools.partial(
    pl.kernel,
    out_type=(
        jax.ShapeDtypeStruct((NC, NPAD), jnp.float32),
        jax.ShapeDtypeStruct((NC, NPAD), jnp.float32),
    ),
    mesh=_mesh,
    scratch_types=[
        pltpu.VMEM((EPT,), jnp.int32),
        pltpu.VMEM((EPT,), jnp.int32),
        pltpu.VMEM((DCH,), jnp.float32),
        pltpu.VMEM((DCH,), jnp.float32),
        pltpu.SemaphoreType.DMA,
        pltpu.SemaphoreType.DMA,
        pltpu.VMEM_SHARED((NPAD,), jnp.float32),
        pltpu.VMEM_SHARED((NPAD,), jnp.float32),
    ],
    compiler_params=pltpu.CompilerParams(use_tc_tiling_on_sc=False, needs_layout_passes=False),
)
def _deg_kernel(srcr, dstr, dego_out, degi_out,
                src_v, dst_v, ones_v, zb_v, sem0, sem1, dego_sp, degi_sp):
    c = lax.axis_index("c")
    s = lax.axis_index("s")
    _fill_1d(ones_v, DCH, 1.0)
    _fill_1d(zb_v, DCH, 0.0)

    base = pl.multiple_of(s * RPT, 8)

    def zloop(k, _):
        off = pl.multiple_of(base + k * DCH, 8)
        pltpu.sync_copy(zb_v, dego_sp.at[pl.ds(off, DCH)])
        pltpu.sync_copy(zb_v, degi_sp.at[pl.ds(off, DCH)])
        return 0

    lax.fori_loop(0, RPT // DCH, zloop, 0)
    plsc.subcore_barrier()

    ebase = pl.multiple_of((c * NS + s) * EPT, 8)
    pltpu.sync_copy(srcr.at[pl.ds(ebase, EPT)], src_v)
    pltpu.sync_copy(dstr.at[pl.ds(ebase, EPT)], dst_v)

    def sloop(j, _):
        pltpu.async_copy(ones_v, dego_sp.at[src_v.at[pl.ds(pl.multiple_of(j * DCH, 8), DCH)]], sem0, add=True)
        pltpu.async_copy(ones_v, degi_sp.at[dst_v.at[pl.ds(pl.multiple_of(j * DCH, 8), DCH)]], sem1, add=True)
        return 0

    lax.fori_loop(0, DNF, sloop, 0)
    ot = ones_v.at[pl.ds(0, DTL)]
    pltpu.async_copy(ot, dego_sp.at[src_v.at[pl.ds(DNF * DCH, DTL)]], sem0, add=True)
    pltpu.async_copy(ot, degi_sp.at[dst_v.at[pl.ds(DNF * DCH, DTL)]], sem1, add=True)

    def dloop(j, _):
        pltpu.make_async_copy(ones_v, dego_sp.at[src_v.at[pl.ds(pl.multiple_of(j * DCH, 8), DCH)]], sem0).wait()
        pltpu.make_async_copy(ones_v, degi_sp.at[dst_v.at[pl.ds(pl.multiple_of(j * DCH, 8), DCH)]], sem1).wait()
        return 0

    lax.fori_loop(0, DNF, dloop, 0)
    pltpu.make_async_copy(ot, dego_sp.at[src_v.at[pl.ds(DNF * DCH, DTL)]], sem0).wait()
    pltpu.make_async_copy(ot, degi_sp.at[dst_v.at[pl.ds(DNF * DCH, DTL)]], sem1).wait()
    plsc.subcore_barrier()

    pltpu.sync_copy(dego_sp.at[pl.ds(base, RPT)], dego_out.at[c, pl.ds(base, RPT)])
    pltpu.sync_copy(degi_sp.at[pl.ds(base, RPT)], degi_out.at[c, pl.ds(base, RPT)])


@functools.partial(
    pl.kernel,
    out_type=jax.ShapeDtypeStruct((NC, NPAD, D_HID), jnp.float32),
    mesh=_mesh,
    scratch_types=[
        pltpu.VMEM((EPT,), jnp.int32),
        pltpu.VMEM((EPT,), jnp.int32),
        [pltpu.VMEM((CH, D_HID), jnp.float32)] * NBUF,
        pltpu.VMEM((CH, D_HID), jnp.float32),
        pltpu.VMEM((RPT, D_HID), jnp.float32),
        pltpu.VMEM((RPT,), jnp.float32),
        pltpu.VMEM((RPT,), jnp.float32),
        [pltpu.SemaphoreType.DMA] * NBUF,
        [pltpu.SemaphoreType.DMA] * NBUF,
        pltpu.VMEM_SHARED((NPAD, D_HID), jnp.float32),
        pltpu.VMEM_SHARED((NPAD, D_HID), jnp.float32),
    ],
    compiler_params=pltpu.CompilerParams(use_tc_tiling_on_sc=False, needs_layout_passes=False),
)
def _agg1_kernel(p_hbm, dego, srcr, dstr, out_hbm,
                 src_v, dst_v, rows, zb_v, pv, da, db, gsems, ssems,
                 q_sp, acc_sp):
    c = lax.axis_index("c")
    s = lax.axis_index("s")
    _fill_2d(zb_v, CH, 0.0)

    base = pl.multiple_of(s * RPT, 8)

    def zloop(k, _):
        off = pl.multiple_of(base + k * CH, 8)
        pltpu.sync_copy(zb_v, acc_sp.at[pl.ds(off, CH)])
        return 0

    lax.fori_loop(0, RPT // CH, zloop, 0)

    pltpu.sync_copy(p_hbm.at[pl.ds(base, RPT)], pv)
    pltpu.sync_copy(dego.at[0, pl.ds(base, RPT)], da)
    pltpu.sync_copy(dego.at[1, pl.ds(base, RPT)], db)

    iota16 = lax.iota(jnp.int32, 16)

    def scale(g, _):
        o = g * 16
        ns16 = _fast_rsqrt(jnp.maximum(da[pl.ds(o, 16)] + db[pl.ds(o, 16)], 1.0))
        rows16 = o + iota16
        for f in range(D_HID):
            fidx = jnp.full((16,), f, jnp.int32)
            col = plsc.load_gather(pv, [rows16, fidx])
            plsc.store_scatter(pv, [rows16, fidx], col * ns16)
        return 0

    lax.fori_loop(0, NGRP, scale, 0)
    pltpu.sync_copy(pv, q_sp.at[pl.ds(base, RPT)])
    plsc.subcore_barrier()

    ebase = pl.multiple_of((c * NS + s) * EPT, 8)
    pltpu.sync_copy(srcr.at[pl.ds(ebase, EPT)], src_v)
    pltpu.sync_copy(dstr.at[pl.ds(ebase, EPT)], dst_v)
    _edge_loop(q_sp, acc_sp, src_v, dst_v, rows, gsems, ssems)
    plsc.subcore_barrier()

    pltpu.sync_copy(acc_sp.at[pl.ds(base, RPT)], out_hbm.at[c, pl.ds(base, RPT)])


@functools.partial(
    pl.kernel,
    out_type=jax.ShapeDtypeStruct((NC, NPAD, D_HID), jnp.float32),
    mesh=_mesh,
    scratch_types=[
        pltpu.VMEM((EPT,), jnp.int32),
        pltpu.VMEM((EPT,), jnp.int32),
        [pltpu.VMEM((CH, D_HID), jnp.float32)] * NBUF,
        pltpu.VMEM((CH, D_HID), jnp.float32),
        pltpu.VMEM((RPT, D_HID), jnp.float32),
        pltpu.VMEM((RPT, D_HID), jnp.float32),
        pltpu.VMEM((RPT,), jnp.float32),
        pltpu.VMEM((RPT,), jnp.float32),
        pltpu.VMEM((RPT,), jnp.float32),
        pltpu.VMEM((16,), jnp.float32),
        [pltpu.SemaphoreType.DMA] * NBUF,
        [pltpu.SemaphoreType.DMA] * NBUF,
        pltpu.VMEM_SHARED((NPAD, D_HID), jnp.float32),
        pltpu.VMEM_SHARED((NPAD, D_HID), jnp.float32),
    ],
    compiler_params=pltpu.CompilerParams(use_tc_tiling_on_sc=False, needs_layout_passes=False),
)
def _agg2_kernel(p1, dego, degi, b1, srcr, dstr, out_hbm,
                 src_v, dst_v, rows, zb_v, p0v, p1v, da, db, dc, b1v,
                 gsems, ssems, q_sp, acc_sp):
    c = lax.axis_index("c")
    s = lax.axis_index("s")
    _fill_2d(zb_v, CH, 0.0)

    base = pl.multiple_of(s * RPT, 8)

    def zloop(k, _):
        off = pl.multiple_of(base + k * CH, 8)
        pltpu.sync_copy(zb_v, acc_sp.at[pl.ds(off, CH)])
        return 0

    lax.fori_loop(0, RPT // CH, zloop, 0)

    pltpu.sync_copy(p1.at[0, pl.ds(base, RPT)], p0v)
    pltpu.sync_copy(p1.at[1, pl.ds(base, RPT)], p1v)
    pltpu.sync_copy(b1, b1v)

    pltpu.sync_copy(dego.at[0, pl.ds(base, RPT)], da)
    pltpu.sync_copy(dego.at[1, pl.ds(base, RPT)], dc)

    def nsl(i, _):
        o = i * 16
        da[pl.ds(o, 16)] = _fast_rsqrt(
            jnp.maximum(da[pl.ds(o, 16)] + dc[pl.ds(o, 16)], 1.0))
        return 0

    lax.fori_loop(0, RPT // 16, nsl, 0)
    pltpu.sync_copy(degi.at[0, pl.ds(base, RPT)], db)
    pltpu.sync_copy(degi.at[1, pl.ds(base, RPT)], dc)

    def ndl(i, _):
        o = i * 16
        db[pl.ds(o, 16)] = _fast_rsqrt(
            jnp.maximum(db[pl.ds(o, 16)] + dc[pl.ds(o, 16)], 1.0))
        return 0

    lax.fori_loop(0, RPT // 16, ndl, 0)

    iota16 = lax.iota(jnp.int32, 16)
    bexp = [plsc.load_gather(b1v, [jnp.full((16,), f, jnp.int32)])
            for f in range(D_HID)]

    def mid(g, _):
        o = g * 16
        ns16 = da[pl.ds(o, 16)]
        nd16 = db[pl.ds(o, 16)]
        rows16 = o + iota16
        for f in range(D_HID):
            fidx = jnp.full((16,), f, jnp.int32)
            g0 = plsc.load_gather(p0v, [rows16, fidx])
            g1 = plsc.load_gather(p1v, [rows16, fidx])
            v = jnp.maximum((g0 + g1) * nd16 + bexp[f], 0.0) * ns16
            plsc.store_scatter(p0v, [rows16, fidx], v)
        return 0

    lax.fori_loop(0, NGRP, mid, 0)
    pltpu.sync_copy(p0v, q_sp.at[pl.ds(base, RPT)])
    plsc.subcore_barrier()

    ebase = pl.multiple_of((c * NS + s) * EPT, 8)
    pltpu.sync_copy(srcr.at[pl.ds(ebase, EPT)], src_v)
    pltpu.sync_copy(dstr.at[pl.ds(ebase, EPT)], dst_v)
    _edge_loop(q_sp, acc_sp, src_v, dst_v, rows, gsems, ssems)
    plsc.subcore_barrier()

    pltpu.sync_copy(acc_sp.at[pl.ds(base, RPT)], out_hbm.at[c, pl.ds(base, RPT)])


def _p_body(x_ref, w_ref, ei_ref, o_ref, os_ref, od_ref):
    p = jnp.dot(x_ref[...], w_ref[...], preferred_element_type=jnp.float32)
    o_ref[...] = jnp.concatenate(
        [p, jnp.zeros((NPAD - N, D_HID), jnp.float32)], axis=0)
    os_ref[...] = ei_ref[0]
    od_ref[...] = ei_ref[1]


_p_call = pl.pallas_call(
    _p_body,
    out_shape=(
        jax.ShapeDtypeStruct((NPAD, D_HID), jnp.float32),
        jax.ShapeDtypeStruct((E,), jnp.int32),
        jax.ShapeDtypeStruct((E,), jnp.int32),
    ))


def _out_body(p_ref, degi_ref, w_ref, b_ref, o_ref):
    agg = p_ref[0] + p_ref[1]
    nd = lax.rsqrt(jnp.maximum(degi_ref[0] + degi_ref[1], 1.0))
    o_ref[...] = jnp.dot(agg * nd[:, None], w_ref[...],
                         preferred_element_type=jnp.float32) + b_ref[...][None, :]


_out_call = pl.pallas_call(
    _out_body, out_shape=jax.ShapeDtypeStruct((NPAD, 40), jnp.float32))


def kernel(x, edge_index, W1, b1, W2, b2):
    ei = edge_index.astype(jnp.int32)

    P, srcp, dstp = _p_call(x, W1, ei)
    dego, degi = _deg_kernel(srcp, dstp)
    p1 = _agg1_kernel(P, dego, srcp, dstp)
    p2 = _agg2_kernel(p1, dego, degi, b1, srcp, dstp)
    out = _out_call(p2, degi, W2, b2)
    return out[:N]
